# baseline (speedup 1.0000x reference)
import functools

import jax
import jax.numpy as jnp
from jax import lax
from jax.experimental import pallas as pl
from jax.experimental.pallas import tpu as pltpu
from jax.experimental.pallas import tpu_sc as plsc

_N = 100000
_D = 128
_B = 1024
_L = 16
_T = 1000
_NT = _N // _T
_NS = _NT // 4
_G = 32

import numpy as _np
_u_const = _np.asarray(jax.random.uniform(jax.random.key(42), (_B, _L)))
_GUMBEL = _np.asarray(-_np.log(-_np.log(_u_const + 1e-20) + 1e-20),
                      _np.float32)


@functools.cache
def _sc_gather_builder(n_rows, d):
    info = plsc.get_sparse_core_info()
    nw = info.num_cores * info.num_subcores
    b_per_w = n_rows // nw
    mesh = plsc.VectorSubcoreMesh(core_axis_name="c", subcore_axis_name="s")

    @functools.partial(
        pl.kernel, mesh=mesh,
        out_type=jax.ShapeDtypeStruct((n_rows, d), jnp.float32),
        scratch_types=[
            pltpu.VMEM((b_per_w,), jnp.int32),
            pltpu.VMEM((b_per_w, d), jnp.float32),
            pltpu.SemaphoreType.DMA,
        ],
    )
    def sc_gather(table_hbm, idx_hbm, out_hbm, idx_v, rows_v, sem):
        wid = lax.axis_index("s") * info.num_cores + lax.axis_index("c")
        base = wid * b_per_w
        pltpu.sync_copy(idx_hbm.at[pl.ds(base, b_per_w)], idx_v)
        pltpu.async_copy(table_hbm.at[idx_v], rows_v, sem).wait()
        pltpu.sync_copy(rows_v, out_hbm.at[pl.ds(base, b_per_w)])

    return sc_gather


def _gather_rows(x, idx):
    return _sc_gather_builder(idx.shape[0], x.shape[1])(x, idx)


def _contrib(emb, gid0, lo, hi):
    gid = gid0 + lax.broadcasted_iota(jnp.int32, (_G, _T), 1)
    onehot = (gid >= lo) & (gid < hi)
    oh = onehot.astype(jnp.bfloat16)
    hi16 = emb.astype(jnp.bfloat16)
    r1 = emb - hi16.astype(jnp.float32)
    mid = r1.astype(jnp.bfloat16)
    lo16 = (r1 - mid.astype(jnp.float32)).astype(jnp.bfloat16)
    dims = (((1,), (0,)), ((), ()))
    return (
        lax.dot_general(oh, hi16, dims, preferred_element_type=jnp.float32)
        + lax.dot_general(oh, mid, dims, preferred_element_type=jnp.float32)
        + lax.dot_general(oh, lo16, dims, preferred_element_type=jnp.float32)
    )


def _pool_body(bases_ref, xa_ref, xb_ref, xc_ref, xd_ref,
               loa_ref, hia_ref, lob_ref, hib_ref,
               loc_ref, hic_ref, lod_ref, hid_ref,
               wcore_ref, bcore_ref, pool_ref):
    t = pl.program_id(0)

    @pl.when(t == 0)
    def _init():
        pool_ref[...] = jnp.zeros_like(pool_ref)

    w = wcore_ref[...]
    b = bcore_ref[...]
    for q, (x_ref, lo_ref, hi_ref) in enumerate([
            (xa_ref, loa_ref, hia_ref), (xb_ref, lob_ref, hib_ref),
            (xc_ref, loc_ref, hic_ref), (xd_ref, lod_ref, hid_ref)]):
        emb = jnp.tanh(jnp.dot(x_ref[...], w,
                               preferred_element_type=jnp.float32) + b)
        c = _contrib(emb, (t + q * _NS) * _T, lo_ref[0], hi_ref[0])
        base = pl.multiple_of(bases_ref[t + q * _NS], 8)
        pool_ref[pl.ds(base, _G), :] += c


def _pooled_emb(x, lo_win, hi_win, bases, w_core, b_core2d):
    grid_spec = pltpu.PrefetchScalarGridSpec(
        num_scalar_prefetch=1,
        grid=(_NS,),
        in_specs=[
            pl.BlockSpec((_T, _D), lambda t, bases: (t, 0)),
            pl.BlockSpec((_T, _D), lambda t, bases: (t + _NS, 0)),
            pl.BlockSpec((_T, _D), lambda t, bases: (t + 2 * _NS, 0)),
            pl.BlockSpec((_T, _D), lambda t, bases: (t + 3 * _NS, 0)),
            pl.BlockSpec((1, _G, 1), lambda t, bases: (t, 0, 0)),
            pl.BlockSpec((1, _G, 1), lambda t, bases: (t, 0, 0)),
            pl.BlockSpec((1, _G, 1), lambda t, bases: (t + _NS, 0, 0)),
            pl.BlockSpec((1, _G, 1), lambda t, bases: (t + _NS, 0, 0)),
            pl.BlockSpec((1, _G, 1), lambda t, bases: (t + 2 * _NS, 0, 0)),
            pl.BlockSpec((1, _G, 1), lambda t, bases: (t + 2 * _NS, 0, 0)),
            pl.BlockSpec((1, _G, 1), lambda t, bases: (t + 3 * _NS, 0, 0)),
            pl.BlockSpec((1, _G, 1), lambda t, bases: (t + 3 * _NS, 0, 0)),
            pl.BlockSpec((_D, _D), lambda t, bases: (0, 0)),
            pl.BlockSpec((1, _D), lambda t, bases: (0, 0)),
        ],
        out_specs=pl.BlockSpec((_B, _D), lambda t, bases: (0, 0)),
    )
    return pl.pallas_call(
        _pool_body,
        grid_spec=grid_spec,
        out_shape=jax.ShapeDtypeStruct((_B, _D), jnp.float32),
    )(bases, x, x, x, x, lo_win, hi_win, lo_win, hi_win,
      lo_win, hi_win, lo_win, hi_win, w_core, b_core2d)


def _head_body(pool_ref, gath_ref, wcore_ref, bcore_ref, wgraph_ref,
               wedge_ref, wlabel_ref, gumbel_ref, logits_ref, labels_ref):
    dot = functools.partial(jnp.dot, preferred_element_type=jnp.float32)
    graph_emb = jnp.tanh(dot(pool_ref[...], wgraph_ref[...]))
    cur = jnp.tanh(dot(gath_ref[0:_B, :], wcore_ref[...]) + bcore_ref[...])
    partner = jnp.tanh(dot(gath_ref[_B:2 * _B, :], wcore_ref[...]) + bcore_ref[...])
    edge_post = jnp.tanh(dot(cur, wedge_ref[0:_D, :])
                         + dot(partner, wedge_ref[_D:2 * _D, :]))
    logits = (dot(edge_post, wlabel_ref[0:_D, :])
              + dot(graph_emb, wlabel_ref[_D:2 * _D, :]))
    logits_ref[...] = logits
    m = jnp.max(logits, axis=1, keepdims=True)
    e = jnp.exp(logits - m)
    probs = e / jnp.sum(e, axis=1, keepdims=True)
    vals = jnp.log(probs + 1e-20) + gumbel_ref[...]
    vmax = jnp.max(vals, axis=1, keepdims=True)
    io = lax.broadcasted_iota(jnp.int32, (_B, _L), 1)
    labels_ref[...] = jnp.min(jnp.where(vals == vmax, io, _L), axis=1,
                              keepdims=True)


def _head(pool, gathered, w_core, b_core2d, w_graph, w_edge, w_label, gumbel):
    return pl.pallas_call(
        _head_body,
        out_shape=(
            jax.ShapeDtypeStruct((_B, _L), jnp.float32),
            jax.ShapeDtypeStruct((_B, 1), jnp.int32),
        ),
    )(pool, gathered, w_core, b_core2d, w_graph, w_edge, w_label, gumbel)


def kernel(x, node_offsets, targets, W_core, b_core, W_graph, W_edge, W_label):
    offs_e = node_offsets.at[-1].set(_N)
    tile_starts = jnp.arange(_NT, dtype=jnp.int32) * _T
    tile_seg = jnp.searchsorted(offs_e, tile_starts,
                                side="right").astype(jnp.int32) - 1
    bases = jnp.clip((tile_seg // 8) * 8, 0, _B - _G).astype(jnp.int32)
    widx = bases[:, None] + jnp.arange(_G, dtype=jnp.int32)[None, :]
    lo_win = offs_e[widx].reshape(_NT, _G, 1)
    hi_win = offs_e[widx + 1].reshape(_NT, _G, 1)
    b_core2d = b_core.reshape(1, _D)

    gumbel = jnp.asarray(_GUMBEL)

    cat_idx = jnp.concatenate([node_offsets[1:] - 1, targets]).astype(jnp.int32)
    gathered = _gather_rows(x, cat_idx)

    pool = _pooled_emb(x, lo_win, hi_win, bases, W_core, b_core2d)

    logits, labels = _head(pool, gathered, W_core, b_core2d, W_graph,
                           W_edge, W_label, gumbel)
    return labels.reshape(_B), logits


# speedup vs baseline: 1.7400x; 1.7400x over previous
import functools

import jax
import jax.numpy as jnp
from jax import lax
from jax.experimental import pallas as pl
from jax.experimental.pallas import tpu as pltpu
from jax.experimental.pallas import tpu_sc as plsc

_N = 100000
_D = 128
_B = 1024
_L = 16
_T = 2000
_NT = _N // _T
_NS = _NT // 2
_G = 32

import numpy as _np
_u_const = _np.asarray(jax.random.uniform(jax.random.key(42), (_B, _L)))
_GUMBEL = _np.asarray(-_np.log(-_np.log(_u_const + 1e-20) + 1e-20),
                      _np.float32)


@functools.cache
def _sc_gather_builder(n_rows, d):
    info = plsc.get_sparse_core_info()
    nw = info.num_cores * info.num_subcores
    b_per_w = n_rows // nw
    mesh = plsc.VectorSubcoreMesh(core_axis_name="c", subcore_axis_name="s")

    @functools.partial(
        pl.kernel, mesh=mesh,
        out_type=jax.ShapeDtypeStruct((n_rows, d), jnp.float32),
        scratch_types=[
            pltpu.VMEM((b_per_w,), jnp.int32),
            pltpu.VMEM((b_per_w, d), jnp.float32),
            pltpu.SemaphoreType.DMA,
        ],
    )
    def sc_gather(table_hbm, idx_hbm, out_hbm, idx_v, rows_v, sem):
        wid = lax.axis_index("s") * info.num_cores + lax.axis_index("c")
        base = wid * b_per_w
        pltpu.sync_copy(idx_hbm.at[pl.ds(base, b_per_w)], idx_v)
        pltpu.async_copy(table_hbm.at[idx_v], rows_v, sem).wait()
        pltpu.sync_copy(rows_v, out_hbm.at[pl.ds(base, b_per_w)])

    return sc_gather


def _gather_rows(x, idx):
    return _sc_gather_builder(idx.shape[0], x.shape[1])(x, idx)


def _contrib(emb, gid0, lo, hi):
    gid = gid0 + lax.broadcasted_iota(jnp.int32, (_G, _T), 1)
    onehot = (gid >= lo) & (gid < hi)
    oh = onehot.astype(jnp.bfloat16)
    hi16 = emb.astype(jnp.bfloat16)
    r1 = emb - hi16.astype(jnp.float32)
    mid = r1.astype(jnp.bfloat16)
    lo16 = (r1 - mid.astype(jnp.float32)).astype(jnp.bfloat16)
    dims = (((1,), (0,)), ((), ()))
    return (
        lax.dot_general(oh, hi16, dims, preferred_element_type=jnp.float32)
        + lax.dot_general(oh, mid, dims, preferred_element_type=jnp.float32)
        + lax.dot_general(oh, lo16, dims, preferred_element_type=jnp.float32)
    )


def _pool_body(bases_ref, xa_ref, xb_ref, loa_ref, hia_ref, lob_ref, hib_ref,
               wcore_ref, bcore_ref, pool_ref):
    t = pl.program_id(0)

    @pl.when(t == 0)
    def _init():
        pool_ref[...] = jnp.zeros_like(pool_ref)

    w = wcore_ref[...]
    b = bcore_ref[...]
    emb_a = jnp.tanh(jnp.dot(xa_ref[...], w,
                             preferred_element_type=jnp.float32) + b)
    emb_b = jnp.tanh(jnp.dot(xb_ref[...], w,
                             preferred_element_type=jnp.float32) + b)
    ca = _contrib(emb_a, t * _T, loa_ref[0], hia_ref[0])
    cb = _contrib(emb_b, (t + _NS) * _T, lob_ref[0], hib_ref[0])
    base_a = pl.multiple_of(bases_ref[t], 8)
    pool_ref[pl.ds(base_a, _G), :] += ca
    base_b = pl.multiple_of(bases_ref[t + _NS], 8)
    pool_ref[pl.ds(base_b, _G), :] += cb


def _pooled_emb(x, lo_win, hi_win, bases, w_core, b_core2d):
    grid_spec = pltpu.PrefetchScalarGridSpec(
        num_scalar_prefetch=1,
        grid=(_NS,),
        in_specs=[
            pl.BlockSpec((_T, _D), lambda t, bases: (t, 0)),
            pl.BlockSpec((_T, _D), lambda t, bases: (t + _NS, 0)),
            pl.BlockSpec((1, _G, 1), lambda t, bases: (t, 0, 0)),
            pl.BlockSpec((1, _G, 1), lambda t, bases: (t, 0, 0)),
            pl.BlockSpec((1, _G, 1), lambda t, bases: (t + _NS, 0, 0)),
            pl.BlockSpec((1, _G, 1), lambda t, bases: (t + _NS, 0, 0)),
            pl.BlockSpec((_D, _D), lambda t, bases: (0, 0)),
            pl.BlockSpec((1, _D), lambda t, bases: (0, 0)),
        ],
        out_specs=pl.BlockSpec((_B, _D), lambda t, bases: (0, 0)),
    )
    return pl.pallas_call(
        _pool_body,
        grid_spec=grid_spec,
        out_shape=jax.ShapeDtypeStruct((_B, _D), jnp.float32),
    )(bases, x, x, lo_win, hi_win, lo_win, hi_win, w_core, b_core2d)


def _head_body(pool_ref, gath_ref, wcore_ref, bcore_ref, wgraph_ref,
               wedge_ref, wlabel_ref, gumbel_ref, logits_ref, labels_ref):
    dot = functools.partial(jnp.dot, preferred_element_type=jnp.float32)
    graph_emb = jnp.tanh(dot(pool_ref[...], wgraph_ref[...]))
    cur = jnp.tanh(dot(gath_ref[0:_B, :], wcore_ref[...]) + bcore_ref[...])
    partner = jnp.tanh(dot(gath_ref[_B:2 * _B, :], wcore_ref[...]) + bcore_ref[...])
    edge_post = jnp.tanh(dot(cur, wedge_ref[0:_D, :])
                         + dot(partner, wedge_ref[_D:2 * _D, :]))
    logits = (dot(edge_post, wlabel_ref[0:_D, :])
              + dot(graph_emb, wlabel_ref[_D:2 * _D, :]))
    logits_ref[...] = logits
    m = jnp.max(logits, axis=1, keepdims=True)
    e = jnp.exp(logits - m)
    probs = e / jnp.sum(e, axis=1, keepdims=True)
    vals = jnp.log(probs + 1e-20) + gumbel_ref[...]
    vmax = jnp.max(vals, axis=1, keepdims=True)
    io = lax.broadcasted_iota(jnp.int32, (_B, _L), 1)
    labels_ref[...] = jnp.min(jnp.where(vals == vmax, io, _L), axis=1,
                              keepdims=True)


def _head(pool, gathered, w_core, b_core2d, w_graph, w_edge, w_label, gumbel):
    return pl.pallas_call(
        _head_body,
        out_shape=(
            jax.ShapeDtypeStruct((_B, _L), jnp.float32),
            jax.ShapeDtypeStruct((_B, 1), jnp.int32),
        ),
    )(pool, gathered, w_core, b_core2d, w_graph, w_edge, w_label, gumbel)


def kernel(x, node_offsets, targets, W_core, b_core, W_graph, W_edge, W_label):
    offs_e = node_offsets.at[-1].set(_N)
    tile_starts = jnp.arange(_NT, dtype=jnp.int32) * _T
    tile_seg = jnp.searchsorted(offs_e, tile_starts,
                                side="right").astype(jnp.int32) - 1
    bases = jnp.clip((tile_seg // 8) * 8, 0, _B - _G).astype(jnp.int32)
    widx = bases[:, None] + jnp.arange(_G, dtype=jnp.int32)[None, :]
    lo_win = offs_e[widx].reshape(_NT, _G, 1)
    hi_win = offs_e[widx + 1].reshape(_NT, _G, 1)
    b_core2d = b_core.reshape(1, _D)

    gumbel = jnp.asarray(_GUMBEL)

    cat_idx = jnp.concatenate([node_offsets[1:] - 1, targets]).astype(jnp.int32)
    gathered = _gather_rows(x, cat_idx)

    pool = _pooled_emb(x, lo_win, hi_win, bases, W_core, b_core2d)

    logits, labels = _head(pool, gathered, W_core, b_core2d, W_graph,
                           W_edge, W_label, gumbel)
    return labels.reshape(_B), logits
